# baseline (speedup 1.0000x reference)
import functools

import jax
import jax.numpy as jnp
from jax.experimental import pallas as pl
from jax.experimental.pallas import tpu as pltpu

_TN = 512


def _moe_body(x_ref, wg_hbm, bg_hbm, w1_hbm, b1_hbm, w2_hbm, b2_hbm, o_ref,
              wg_s, bg_s, w1_s, b1_s, w2r_s, b2_s, w1t_s, b1r_s, sems,
              *, n_exp, f_hid):
    i = pl.program_id(0)
    ef = n_exp * f_hid

    @pl.when(i == 0)
    def _load_weights():
        copies = [
            pltpu.make_async_copy(wg_hbm, wg_s, sems.at[0]),
            pltpu.make_async_copy(bg_hbm, bg_s, sems.at[1]),
            pltpu.make_async_copy(w1_hbm, w1_s, sems.at[2]),
            pltpu.make_async_copy(b1_hbm, b1_s, sems.at[3]),
            pltpu.make_async_copy(b2_hbm, b2_s, sems.at[4]),
        ]
        for e in range(n_exp):
            copies.append(pltpu.make_async_copy(
                w2_hbm.at[e], w2r_s.at[e * f_hid:(e + 1) * f_hid, :],
                sems.at[5 + e]))
        for c in copies:
            c.start()
        for c in copies:
            c.wait()
        w1t_s[...] = jnp.concatenate(
            [w1_s[e] for e in range(n_exp)], axis=1).astype(jnp.bfloat16)
        b1r_s[...] = jnp.concatenate(
            [b1_s[e] for e in range(n_exp)], axis=0).reshape(1, ef)

    x = x_ref[...]
    xb = x.astype(jnp.bfloat16)
    logits = jnp.dot(xb, wg_s[...].astype(jnp.bfloat16),
                     preferred_element_type=jnp.float32)
    logits = logits + bg_s[...][None, :]
    m = jnp.max(logits, axis=1, keepdims=True)
    p = jnp.exp(logits - m)
    g = p / jnp.sum(p, axis=1, keepdims=True)
    gb = g.astype(jnp.bfloat16)

    h = jnp.dot(xb, w1t_s[...], preferred_element_type=jnp.float32)
    h = jnp.maximum(h + b1r_s[...], 0.0).astype(jnp.bfloat16)

    rr = jax.lax.broadcasted_iota(jnp.int32, (n_exp, ef), 0)
    cc = jax.lax.broadcasted_iota(jnp.int32, (n_exp, ef), 1)
    sel = (cc // f_hid == rr).astype(jnp.bfloat16)
    ge = jnp.dot(gb, sel,
                 preferred_element_type=jnp.float32).astype(jnp.bfloat16)

    out = jnp.dot(h * ge, w2r_s[...].astype(jnp.bfloat16),
                  preferred_element_type=jnp.float32)
    out = out + jnp.dot(gb, b2_s[...].astype(jnp.bfloat16),
                        preferred_element_type=jnp.float32)
    o_ref[...] = out


def kernel(x, Wg, bg, W1, b1, W2, b2):
    n, d = x.shape
    e, _, f = W1.shape
    ef = e * f
    any_spec = pl.BlockSpec(memory_space=pl.ANY)
    return pl.pallas_call(
        functools.partial(_moe_body, n_exp=e, f_hid=f),
        grid=(n // _TN,),
        in_specs=[
            pl.BlockSpec((_TN, d), lambda i: (i, 0)),
            any_spec, any_spec, any_spec, any_spec, any_spec, any_spec,
        ],
        out_specs=pl.BlockSpec((_TN, d), lambda i: (i, 0)),
        out_shape=jax.ShapeDtypeStruct((n, d), x.dtype),
        scratch_shapes=[
            pltpu.VMEM((d, e), jnp.float32),
            pltpu.VMEM((e,), jnp.float32),
            pltpu.VMEM((e, d, f), jnp.float32),
            pltpu.VMEM((e, f), jnp.float32),
            pltpu.VMEM((ef, d), jnp.float32),
            pltpu.VMEM((e, d), jnp.float32),
            pltpu.VMEM((d, ef), jnp.bfloat16),
            pltpu.VMEM((1, ef), jnp.float32),
            pltpu.SemaphoreType.DMA((5 + e,)),
        ],
        compiler_params=pltpu.CompilerParams(
            dimension_semantics=("arbitrary",)),
    )(x, Wg, bg, W1, b1, W2, b2)


# speedup vs baseline: 1.0643x; 1.0643x over previous
import functools

import jax
import jax.numpy as jnp
from jax.experimental import pallas as pl
from jax.experimental.pallas import tpu as pltpu

_TN = 2048


def _moe_body(x_ref, wg_hbm, bg_hbm, w1_hbm, b1_hbm, w2_hbm, b2_hbm, o_ref,
              wg_s, bg_s, w1_s, b1_s, w2r_s, b2_s, w1t_s, b1r_s, sems,
              *, n_exp, f_hid):
    i = pl.program_id(0)
    ef = n_exp * f_hid

    @pl.when(i == 0)
    def _load_weights():
        copies = [
            pltpu.make_async_copy(wg_hbm, wg_s, sems.at[0]),
            pltpu.make_async_copy(bg_hbm, bg_s, sems.at[1]),
            pltpu.make_async_copy(w1_hbm, w1_s, sems.at[2]),
            pltpu.make_async_copy(b1_hbm, b1_s, sems.at[3]),
            pltpu.make_async_copy(b2_hbm, b2_s, sems.at[4]),
        ]
        for e in range(n_exp):
            copies.append(pltpu.make_async_copy(
                w2_hbm.at[e], w2r_s.at[e * f_hid:(e + 1) * f_hid, :],
                sems.at[5 + e]))
        for c in copies:
            c.start()
        for c in copies:
            c.wait()
        w1t_s[...] = jnp.concatenate(
            [w1_s[e] for e in range(n_exp)], axis=1).astype(jnp.bfloat16)
        b1r_s[...] = jnp.concatenate(
            [b1_s[e] for e in range(n_exp)], axis=0).reshape(1, ef)

    x = x_ref[...]
    xb = x.astype(jnp.bfloat16)
    logits = jnp.dot(xb, wg_s[...].astype(jnp.bfloat16),
                     preferred_element_type=jnp.float32)
    logits = logits + bg_s[...][None, :]
    m = jnp.max(logits, axis=1, keepdims=True)
    p = jnp.exp(logits - m)
    g = p / jnp.sum(p, axis=1, keepdims=True)
    gb = g.astype(jnp.bfloat16)

    h = jnp.dot(xb, w1t_s[...], preferred_element_type=jnp.float32)
    h = jnp.maximum(h + b1r_s[...], 0.0).astype(jnp.bfloat16)

    rr = jax.lax.broadcasted_iota(jnp.int32, (n_exp, ef), 0)
    cc = jax.lax.broadcasted_iota(jnp.int32, (n_exp, ef), 1)
    sel = (cc // f_hid == rr).astype(jnp.bfloat16)
    ge = jnp.dot(gb, sel,
                 preferred_element_type=jnp.float32).astype(jnp.bfloat16)

    out = jnp.dot(h * ge, w2r_s[...].astype(jnp.bfloat16),
                  preferred_element_type=jnp.float32)
    out = out + jnp.dot(gb, b2_s[...].astype(jnp.bfloat16),
                        preferred_element_type=jnp.float32)
    o_ref[...] = out


def kernel(x, Wg, bg, W1, b1, W2, b2):
    n, d = x.shape
    e, _, f = W1.shape
    ef = e * f
    any_spec = pl.BlockSpec(memory_space=pl.ANY)
    return pl.pallas_call(
        functools.partial(_moe_body, n_exp=e, f_hid=f),
        grid=(n // _TN,),
        in_specs=[
            pl.BlockSpec((_TN, d), lambda i: (i, 0)),
            any_spec, any_spec, any_spec, any_spec, any_spec, any_spec,
        ],
        out_specs=pl.BlockSpec((_TN, d), lambda i: (i, 0)),
        out_shape=jax.ShapeDtypeStruct((n, d), x.dtype),
        scratch_shapes=[
            pltpu.VMEM((d, e), jnp.float32),
            pltpu.VMEM((e,), jnp.float32),
            pltpu.VMEM((e, d, f), jnp.float32),
            pltpu.VMEM((e, f), jnp.float32),
            pltpu.VMEM((ef, d), jnp.float32),
            pltpu.VMEM((e, d), jnp.float32),
            pltpu.VMEM((d, ef), jnp.bfloat16),
            pltpu.VMEM((1, ef), jnp.float32),
            pltpu.SemaphoreType.DMA((5 + e,)),
        ],
        compiler_params=pltpu.CompilerParams(
            dimension_semantics=("arbitrary",)),
    )(x, Wg, bg, W1, b1, W2, b2)


# speedup vs baseline: 1.0776x; 1.0124x over previous
import functools

import jax
import jax.numpy as jnp
from jax.experimental import pallas as pl
from jax.experimental.pallas import tpu as pltpu

_TN = 1024


def _moe_body(x_ref, wg_hbm, bg_hbm, w1_hbm, b1_hbm, w2_hbm, b2_hbm, o_ref,
              wg_s, bg_s, w1_s, b1_s, w2r_s, b2_s, w1t_s, b1r_s, sems,
              *, n_exp, f_hid):
    i = pl.program_id(0)
    ef = n_exp * f_hid

    @pl.when(i == 0)
    def _load_weights():
        copies = [
            pltpu.make_async_copy(wg_hbm, wg_s, sems.at[0]),
            pltpu.make_async_copy(bg_hbm, bg_s, sems.at[1]),
            pltpu.make_async_copy(w1_hbm, w1_s, sems.at[2]),
            pltpu.make_async_copy(b1_hbm, b1_s, sems.at[3]),
            pltpu.make_async_copy(b2_hbm, b2_s, sems.at[4]),
        ]
        for e in range(n_exp):
            copies.append(pltpu.make_async_copy(
                w2_hbm.at[e], w2r_s.at[e * f_hid:(e + 1) * f_hid, :],
                sems.at[5 + e]))
        for c in copies:
            c.start()
        for c in copies:
            c.wait()
        w1t_s[...] = jnp.concatenate(
            [w1_s[e] for e in range(n_exp)], axis=1).astype(jnp.bfloat16)
        b1r_s[...] = jnp.concatenate(
            [b1_s[e] for e in range(n_exp)], axis=0).reshape(1, ef)

    x = x_ref[...]
    xb = x.astype(jnp.bfloat16)
    logits = jnp.dot(xb, wg_s[...].astype(jnp.bfloat16),
                     preferred_element_type=jnp.float32)
    logits = logits + bg_s[...][None, :]
    m = jnp.max(logits, axis=1, keepdims=True)
    p = jnp.exp(logits - m)
    g = p / jnp.sum(p, axis=1, keepdims=True)
    gb = g.astype(jnp.bfloat16)

    h = jnp.dot(xb, w1t_s[...], preferred_element_type=jnp.float32)
    h = jnp.maximum(h + b1r_s[...], 0.0).astype(jnp.bfloat16)

    rr = jax.lax.broadcasted_iota(jnp.int32, (n_exp, ef), 0)
    cc = jax.lax.broadcasted_iota(jnp.int32, (n_exp, ef), 1)
    sel = (cc // f_hid == rr).astype(jnp.bfloat16)
    ge = jnp.dot(gb, sel,
                 preferred_element_type=jnp.float32).astype(jnp.bfloat16)

    out = jnp.dot(h * ge, w2r_s[...].astype(jnp.bfloat16),
                  preferred_element_type=jnp.float32)
    out = out + jnp.dot(gb, b2_s[...].astype(jnp.bfloat16),
                        preferred_element_type=jnp.float32)
    o_ref[...] = out


def kernel(x, Wg, bg, W1, b1, W2, b2):
    n, d = x.shape
    e, _, f = W1.shape
    ef = e * f
    any_spec = pl.BlockSpec(memory_space=pl.ANY)
    return pl.pallas_call(
        functools.partial(_moe_body, n_exp=e, f_hid=f),
        grid=(n // _TN,),
        in_specs=[
            pl.BlockSpec((_TN, d), lambda i: (i, 0)),
            any_spec, any_spec, any_spec, any_spec, any_spec, any_spec,
        ],
        out_specs=pl.BlockSpec((_TN, d), lambda i: (i, 0)),
        out_shape=jax.ShapeDtypeStruct((n, d), x.dtype),
        scratch_shapes=[
            pltpu.VMEM((d, e), jnp.float32),
            pltpu.VMEM((e,), jnp.float32),
            pltpu.VMEM((e, d, f), jnp.float32),
            pltpu.VMEM((e, f), jnp.float32),
            pltpu.VMEM((ef, d), jnp.float32),
            pltpu.VMEM((e, d), jnp.float32),
            pltpu.VMEM((d, ef), jnp.bfloat16),
            pltpu.VMEM((1, ef), jnp.float32),
            pltpu.SemaphoreType.DMA((5 + e,)),
        ],
        compiler_params=pltpu.CompilerParams(
            dimension_semantics=("arbitrary",)),
    )(x, Wg, bg, W1, b1, W2, b2)
